# baseline (speedup 1.0000x reference)
import functools

import jax
import jax.numpy as jnp
from jax import lax
from jax.experimental import pallas as pl
from jax.experimental.pallas import tpu as pltpu
from jax.experimental.pallas import tpu_sc as plsc

EMB_DIM = 64
CTX = 10
B_POS = 16384
B_NEG = 81920
B_TOT = B_POS + B_NEG
TROWS = 199999
NC = 2
NS = 16
NW = NC * NS
POS_PER_W = B_POS // NW
NEG_PER_W = B_NEG // NW
C = 32
NPOS_CHUNK = POS_PER_W // C
NNEG_CHUNK = NEG_PER_W // C
NCHUNK = NPOS_CHUNK + NNEG_CHUNK
ITEMS_PER_W = POS_PER_W + NEG_PER_W
NBUF = 2
ACC_LEN = B_TOT * EMB_DIM
OUT_LEN = B_TOT * 16
R128 = OUT_LEN // 128
POS_CTX_LEN = POS_PER_W * CTX


def _prep_idx_body(pvt_ref, nvt_ref, opv_ref, onv_ref):
    opv_ref[...] = jnp.pad(pvt_ref[...], ((0, 16 - CTX), (0, 0)))
    onv_ref[...] = jnp.pad(nvt_ref[...], ((0, 16 - CTX), (0, 0)))


_prep_idx = pl.pallas_call(
    _prep_idx_body,
    out_shape=[
        jax.ShapeDtypeStruct((16, B_POS), jnp.int32),
        jax.ShapeDtypeStruct((16, B_NEG), jnp.int32),
    ],
)


def _chunk_off(wid, j):
    return jnp.where(
        j < NPOS_CHUNK,
        wid * POS_PER_W + j * C,
        B_POS + wid * NEG_PER_W + (j - NPOS_CHUNK) * C)


def _sc_ctx_body(u_emb, pvt, nvt, out,
                 idx_v_all, rows_v0, rows_v1, parts0, parts1,
                 semg0, semg1, semo0, semo1):
    rows_v = (rows_v0, rows_v1)
    parts = (parts0, parts1)
    semg = (semg0, semg1)
    semo = (semo0, semo1)

    wid = lax.axis_index("s") * NC + lax.axis_index("c")
    idx_cps = []
    for c in range(CTX):
        idx_cps.append(pltpu.async_copy(
            pvt.at[c, pl.ds(wid * POS_PER_W, POS_PER_W)],
            idx_v_all.at[pl.ds(c * POS_PER_W, POS_PER_W)], semg0))
        idx_cps.append(pltpu.async_copy(
            nvt.at[c, pl.ds(wid * NEG_PER_W, NEG_PER_W)],
            idx_v_all.at[pl.ds(POS_CTX_LEN + c * NEG_PER_W, NEG_PER_W)],
            semg0))
    for cp in idx_cps:
        cp.wait()

    def issue(j, b):
        jp = jnp.minimum(j, NPOS_CHUNK - 1)
        jn_ = jnp.maximum(j - NPOS_CHUNK, 0)
        is_pos = j < NPOS_CHUNK
        for c in range(CTX):
            off = jnp.where(is_pos, c * POS_PER_W + jp * C,
                            POS_CTX_LEN + c * NEG_PER_W + jn_ * C)
            pltpu.async_copy(
                u_emb.at[idx_v_all.at[pl.ds(off, C)]],
                rows_v[b].at[pl.ds(c * C, C)], semg[b])

    def drain_gathers(b):
        pltpu.make_async_copy(
            u_emb.at[pl.ds(0, C * CTX)], rows_v[b], semg[b]).wait()

    def compute(b):
        rv, pt = rows_v[b], parts[b]

        def item_body(i, carry):
            for d in range(4):
                acc = rv[i, pl.ds(d * 16, 16)]
                for c in range(1, CTX):
                    acc = acc + rv[c * C + i, pl.ds(d * 16, 16)]
                pt[pl.ds(i * EMB_DIM + d * 16, 16)] = acc
            return carry

        lax.fori_loop(0, C, item_body, 0)

    issue(0, 0)

    def outer(g, carry):
        for b in range(NBUF):
            j = g * NBUF + b
            jn = j + 1

            @pl.when(jn < NCHUNK)
            def _():
                issue(jn, b ^ 1)

            drain_gathers(b)

            @pl.when(j >= NBUF)
            def _():
                pltpu.make_async_copy(
                    parts[b], out.at[pl.ds(0, C * EMB_DIM)], semo[b]).wait()

            compute(b)
            off = _chunk_off(wid, j)
            pltpu.async_copy(
                parts[b], out.at[pl.ds(off * EMB_DIM, C * EMB_DIM)], semo[b])
        return carry

    lax.fori_loop(0, NCHUNK // NBUF, outer, 0)
    for b in range(NBUF):
        pltpu.make_async_copy(
            parts[b], out.at[pl.ds(0, C * EMB_DIM)], semo[b]).wait()


_sc_ctx = functools.partial(
    pl.kernel,
    out_type=jax.ShapeDtypeStruct((ACC_LEN,), jnp.float32),
    mesh=plsc.VectorSubcoreMesh(core_axis_name="c", subcore_axis_name="s"),
    scratch_types=[
        pltpu.VMEM((ITEMS_PER_W * CTX,), jnp.int32),
        pltpu.VMEM((C * CTX, EMB_DIM), jnp.float32),
        pltpu.VMEM((C * CTX, EMB_DIM), jnp.float32),
        pltpu.VMEM((C * EMB_DIM,), jnp.float32),
        pltpu.VMEM((C * EMB_DIM,), jnp.float32),
        pltpu.SemaphoreType.DMA,
        pltpu.SemaphoreType.DMA,
        pltpu.SemaphoreType.DMA,
        pltpu.SemaphoreType.DMA,
    ],
    compiler_params=pltpu.CompilerParams(use_tc_tiling_on_sc=False),
)(_sc_ctx_body)


def _sc_dot_body(v_emb, pos_u, neg_u, accs, out,
                 idx_u_all, acc_v0, acc_v1, rows_u0, rows_u1,
                 parts0, parts1, semg0, semg1, semo0, semo1):
    acc_v = (acc_v0, acc_v1)
    rows_u = (rows_u0, rows_u1)
    parts = (parts0, parts1)
    semg = (semg0, semg1)
    semo = (semo0, semo1)

    wid = lax.axis_index("s") * NC + lax.axis_index("c")
    cp0 = pltpu.async_copy(pos_u.at[pl.ds(wid * POS_PER_W, POS_PER_W)],
                           idx_u_all.at[pl.ds(0, POS_PER_W)], semg0)
    cp1 = pltpu.async_copy(neg_u.at[pl.ds(wid * NEG_PER_W, NEG_PER_W)],
                           idx_u_all.at[pl.ds(POS_PER_W, NEG_PER_W)], semg0)
    cp0.wait()
    cp1.wait()

    def issue(j, b):
        pltpu.async_copy(
            v_emb.at[idx_u_all.at[pl.ds(j * C, C)]], rows_u[b], semg[b])
        off = _chunk_off(wid, j)
        pltpu.async_copy(
            accs.at[pl.ds(off * EMB_DIM, C * EMB_DIM)], acc_v[b], semg[b])

    def drain_gathers(b):
        pltpu.make_async_copy(v_emb.at[pl.ds(0, C)], rows_u[b], semg[b]).wait()
        pltpu.make_async_copy(
            accs.at[pl.ds(0, C * EMB_DIM)], acc_v[b], semg[b]).wait()

    def compute(b):
        av, ru, pt = acc_v[b], rows_u[b], parts[b]

        def item_body(i, carry):
            part = av[pl.ds(i * EMB_DIM, 16)] * ru[i, pl.ds(0, 16)]
            for d in range(1, 4):
                part = part + (av[pl.ds(i * EMB_DIM + d * 16, 16)]
                               * ru[i, pl.ds(d * 16, 16)])
            pt[pl.ds(i * 16, 16)] = part
            return carry

        lax.fori_loop(0, C, item_body, 0)

    issue(0, 0)

    def outer(g, carry):
        for b in range(NBUF):
            j = g * NBUF + b
            jn = j + 1

            @pl.when(jn < NCHUNK)
            def _():
                issue(jn, b ^ 1)

            drain_gathers(b)

            @pl.when(j >= NBUF)
            def _():
                pltpu.make_async_copy(
                    parts[b], out.at[pl.ds(0, C * 16)], semo[b]).wait()

            compute(b)
            off = _chunk_off(wid, j)
            pltpu.async_copy(
                parts[b], out.at[pl.ds(off * 16, C * 16)], semo[b])
        return carry

    lax.fori_loop(0, NCHUNK // NBUF, outer, 0)
    for b in range(NBUF):
        pltpu.make_async_copy(
            parts[b], out.at[pl.ds(0, C * 16)], semo[b]).wait()


_sc_dot = functools.partial(
    pl.kernel,
    out_type=jax.ShapeDtypeStruct((OUT_LEN,), jnp.float32),
    mesh=plsc.VectorSubcoreMesh(core_axis_name="c", subcore_axis_name="s"),
    scratch_types=[
        pltpu.VMEM((ITEMS_PER_W,), jnp.int32),
        pltpu.VMEM((C * EMB_DIM,), jnp.float32),
        pltpu.VMEM((C * EMB_DIM,), jnp.float32),
        pltpu.VMEM((C, EMB_DIM), jnp.float32),
        pltpu.VMEM((C, EMB_DIM), jnp.float32),
        pltpu.VMEM((C * 16,), jnp.float32),
        pltpu.VMEM((C * 16,), jnp.float32),
        pltpu.SemaphoreType.DMA,
        pltpu.SemaphoreType.DMA,
        pltpu.SemaphoreType.DMA,
        pltpu.SemaphoreType.DMA,
    ],
    compiler_params=pltpu.CompilerParams(use_tc_tiling_on_sc=False),
)(_sc_dot_body)


def _tc_body(parts_ref, o_ref):
    x = parts_ref[...]
    sel = (lax.broadcasted_iota(jnp.int32, (128, 8), 0) // 16
           == lax.broadcasted_iota(jnp.int32, (128, 8), 1)).astype(jnp.float32)
    s = jnp.dot(x, sel, preferred_element_type=jnp.float32)
    row = lax.broadcasted_iota(jnp.int32, (R128, 8), 0)
    sign = jnp.where(row < B_POS // 8, 1.0, -1.0)
    t = s * sign
    ls = jnp.minimum(t, 0.0) - jnp.log(1.0 + jnp.exp(-jnp.abs(t)))
    o_ref[0, 0] = -jnp.sum(ls)


_tc_finish = pl.pallas_call(
    _tc_body,
    out_shape=jax.ShapeDtypeStruct((1, 1), jnp.float32),
    out_specs=pl.BlockSpec(memory_space=pltpu.SMEM),
)


def kernel(pos_u, pos_v, neg_u, neg_v, u_emb, v_emb):
    pvt, nvt = _prep_idx(pos_v.astype(jnp.int32).T, neg_v.astype(jnp.int32).T)
    accs = _sc_ctx(u_emb, pvt, nvt)
    parts = _sc_dot(v_emb, pos_u.astype(jnp.int32), neg_u.astype(jnp.int32),
                    accs)
    loss = _tc_finish(parts.reshape(R128, 128))
    return loss[0, 0]


# speedup vs baseline: 1.0709x; 1.0709x over previous
import functools

import jax
import jax.numpy as jnp
from jax import lax
from jax.experimental import pallas as pl
from jax.experimental.pallas import tpu as pltpu
from jax.experimental.pallas import tpu_sc as plsc

EMB_DIM = 64
CTX = 10
B_POS = 16384
B_NEG = 81920
B_TOT = B_POS + B_NEG
TROWS = 199999
NC = 2
NS = 16
NW = NC * NS
POS_PER_W = B_POS // NW
NEG_PER_W = B_NEG // NW
CA = 64
NPOS_CHUNK_A = POS_PER_W // CA
NNEG_CHUNK_A = NEG_PER_W // CA
NCHUNK_A = NPOS_CHUNK_A + NNEG_CHUNK_A
CB = 128
NPOS_CHUNK_B = POS_PER_W // CB
NNEG_CHUNK_B = NEG_PER_W // CB
NCHUNK_B = NPOS_CHUNK_B + NNEG_CHUNK_B
ITEMS_PER_W = POS_PER_W + NEG_PER_W
NBUF = 2
ACC_LEN = B_TOT * EMB_DIM
OUT_LEN = B_TOT * 16
R128 = OUT_LEN // 128
POS_CTX_LEN = POS_PER_W * CTX


def _prep_idx_body(pvt_ref, nvt_ref, opv_ref, onv_ref):
    opv_ref[...] = jnp.pad(pvt_ref[...], ((0, 16 - CTX), (0, 0)))
    onv_ref[...] = jnp.pad(nvt_ref[...], ((0, 16 - CTX), (0, 0)))


_prep_idx = pl.pallas_call(
    _prep_idx_body,
    out_shape=[
        jax.ShapeDtypeStruct((16, B_POS), jnp.int32),
        jax.ShapeDtypeStruct((16, B_NEG), jnp.int32),
    ],
)


def _chunk_off(wid, j, c_sz, npos_chunk):
    return jnp.where(
        j < npos_chunk,
        wid * POS_PER_W + j * c_sz,
        B_POS + wid * NEG_PER_W + (j - npos_chunk) * c_sz)


def _sc_ctx_body(u_emb, pvt, nvt, out,
                 idx_v_all, rows_v0, rows_v1, parts0, parts1,
                 semg0, semg1, semo0, semo1):
    rows_v = (rows_v0, rows_v1)
    parts = (parts0, parts1)
    semg = (semg0, semg1)
    semo = (semo0, semo1)

    wid = lax.axis_index("s") * NC + lax.axis_index("c")
    idx_cps = []
    for c in range(CTX):
        idx_cps.append(pltpu.async_copy(
            pvt.at[c, pl.ds(wid * POS_PER_W, POS_PER_W)],
            idx_v_all.at[pl.ds(c * POS_PER_W, POS_PER_W)], semg0))
        idx_cps.append(pltpu.async_copy(
            nvt.at[c, pl.ds(wid * NEG_PER_W, NEG_PER_W)],
            idx_v_all.at[pl.ds(POS_CTX_LEN + c * NEG_PER_W, NEG_PER_W)],
            semg0))
    for cp in idx_cps:
        cp.wait()

    def issue(j, b):
        jp = jnp.minimum(j, NPOS_CHUNK_A - 1)
        jn_ = jnp.maximum(j - NPOS_CHUNK_A, 0)
        is_pos = j < NPOS_CHUNK_A
        for c in range(CTX):
            off = jnp.where(is_pos, c * POS_PER_W + jp * CA,
                            POS_CTX_LEN + c * NEG_PER_W + jn_ * CA)
            pltpu.async_copy(
                u_emb.at[idx_v_all.at[pl.ds(off, CA)]],
                rows_v[b].at[pl.ds(c * CA, CA)], semg[b])

    def drain_gathers(b):
        pltpu.make_async_copy(
            u_emb.at[pl.ds(0, CA * CTX)], rows_v[b], semg[b]).wait()

    def compute(b):
        rv, pt = rows_v[b], parts[b]

        def item_body(i, carry):
            for d in range(4):
                acc = rv[i, pl.ds(d * 16, 16)]
                for c in range(1, CTX):
                    acc = acc + rv[c * CA + i, pl.ds(d * 16, 16)]
                pt[pl.ds(i * EMB_DIM + d * 16, 16)] = acc
            return carry

        lax.fori_loop(0, CA, item_body, 0)

    issue(0, 0)

    def outer(g, carry):
        for b in range(NBUF):
            j = g * NBUF + b
            jn = j + 1

            @pl.when(jn < NCHUNK_A)
            def _():
                issue(jn, b ^ 1)

            drain_gathers(b)

            @pl.when(j >= NBUF)
            def _():
                pltpu.make_async_copy(
                    parts[b], out.at[pl.ds(0, CA * EMB_DIM)], semo[b]).wait()

            compute(b)
            off = _chunk_off(wid, j, CA, NPOS_CHUNK_A)
            pltpu.async_copy(
                parts[b], out.at[pl.ds(off * EMB_DIM, CA * EMB_DIM)], semo[b])
        return carry

    lax.fori_loop(0, NCHUNK_A // NBUF, outer, 0)
    for b in range(NBUF):
        pltpu.make_async_copy(
            parts[b], out.at[pl.ds(0, CA * EMB_DIM)], semo[b]).wait()


_sc_ctx = functools.partial(
    pl.kernel,
    out_type=jax.ShapeDtypeStruct((ACC_LEN,), jnp.float32),
    mesh=plsc.VectorSubcoreMesh(core_axis_name="c", subcore_axis_name="s"),
    scratch_types=[
        pltpu.VMEM((ITEMS_PER_W * CTX,), jnp.int32),
        pltpu.VMEM((CA * CTX, EMB_DIM), jnp.float32),
        pltpu.VMEM((CA * CTX, EMB_DIM), jnp.float32),
        pltpu.VMEM((CA * EMB_DIM,), jnp.float32),
        pltpu.VMEM((CA * EMB_DIM,), jnp.float32),
        pltpu.SemaphoreType.DMA,
        pltpu.SemaphoreType.DMA,
        pltpu.SemaphoreType.DMA,
        pltpu.SemaphoreType.DMA,
    ],
    compiler_params=pltpu.CompilerParams(use_tc_tiling_on_sc=False),
)(_sc_ctx_body)


def _sc_dot_body(v_emb, pos_u, neg_u, accs, out,
                 idx_u_all, acc_v0, acc_v1, rows_u0, rows_u1,
                 parts0, parts1, semg0, semg1, semo0, semo1):
    acc_v = (acc_v0, acc_v1)
    rows_u = (rows_u0, rows_u1)
    parts = (parts0, parts1)
    semg = (semg0, semg1)
    semo = (semo0, semo1)

    wid = lax.axis_index("s") * NC + lax.axis_index("c")
    cp0 = pltpu.async_copy(pos_u.at[pl.ds(wid * POS_PER_W, POS_PER_W)],
                           idx_u_all.at[pl.ds(0, POS_PER_W)], semg0)
    cp1 = pltpu.async_copy(neg_u.at[pl.ds(wid * NEG_PER_W, NEG_PER_W)],
                           idx_u_all.at[pl.ds(POS_PER_W, NEG_PER_W)], semg0)
    cp0.wait()
    cp1.wait()

    def issue(j, b):
        pltpu.async_copy(
            v_emb.at[idx_u_all.at[pl.ds(j * CB, CB)]], rows_u[b], semg[b])
        off = _chunk_off(wid, j, CB, NPOS_CHUNK_B)
        pltpu.async_copy(
            accs.at[pl.ds(off * EMB_DIM, CB * EMB_DIM)], acc_v[b], semg[b])

    def drain_gathers(b):
        pltpu.make_async_copy(v_emb.at[pl.ds(0, CB)], rows_u[b], semg[b]).wait()
        pltpu.make_async_copy(
            accs.at[pl.ds(0, CB * EMB_DIM)], acc_v[b], semg[b]).wait()

    def compute(b):
        av, ru, pt = acc_v[b], rows_u[b], parts[b]

        def item_body(i, carry):
            part = av[pl.ds(i * EMB_DIM, 16)] * ru[i, pl.ds(0, 16)]
            for d in range(1, 4):
                part = part + (av[pl.ds(i * EMB_DIM + d * 16, 16)]
                               * ru[i, pl.ds(d * 16, 16)])
            pt[pl.ds(i * 16, 16)] = part
            return carry

        lax.fori_loop(0, CB, item_body, 0)

    issue(0, 0)

    def outer(g, carry):
        for b in range(NBUF):
            j = g * NBUF + b
            jn = j + 1

            @pl.when(jn < NCHUNK_B)
            def _():
                issue(jn, b ^ 1)

            drain_gathers(b)

            @pl.when(j >= NBUF)
            def _():
                pltpu.make_async_copy(
                    parts[b], out.at[pl.ds(0, CB * 16)], semo[b]).wait()

            compute(b)
            off = _chunk_off(wid, j, CB, NPOS_CHUNK_B)
            pltpu.async_copy(
                parts[b], out.at[pl.ds(off * 16, CB * 16)], semo[b])
        return carry

    lax.fori_loop(0, NCHUNK_B // NBUF, outer, 0)
    for b in range(NBUF):
        pltpu.make_async_copy(
            parts[b], out.at[pl.ds(0, CB * 16)], semo[b]).wait()


_sc_dot = functools.partial(
    pl.kernel,
    out_type=jax.ShapeDtypeStruct((OUT_LEN,), jnp.float32),
    mesh=plsc.VectorSubcoreMesh(core_axis_name="c", subcore_axis_name="s"),
    scratch_types=[
        pltpu.VMEM((ITEMS_PER_W,), jnp.int32),
        pltpu.VMEM((CB * EMB_DIM,), jnp.float32),
        pltpu.VMEM((CB * EMB_DIM,), jnp.float32),
        pltpu.VMEM((CB, EMB_DIM), jnp.float32),
        pltpu.VMEM((CB, EMB_DIM), jnp.float32),
        pltpu.VMEM((CB * 16,), jnp.float32),
        pltpu.VMEM((CB * 16,), jnp.float32),
        pltpu.SemaphoreType.DMA,
        pltpu.SemaphoreType.DMA,
        pltpu.SemaphoreType.DMA,
        pltpu.SemaphoreType.DMA,
    ],
    compiler_params=pltpu.CompilerParams(use_tc_tiling_on_sc=False),
)(_sc_dot_body)


def _tc_body(parts_ref, o_ref):
    x = parts_ref[...]
    sel = (lax.broadcasted_iota(jnp.int32, (128, 8), 0) // 16
           == lax.broadcasted_iota(jnp.int32, (128, 8), 1)).astype(jnp.float32)
    s = jnp.dot(x, sel, preferred_element_type=jnp.float32)
    row = lax.broadcasted_iota(jnp.int32, (R128, 8), 0)
    sign = jnp.where(row < B_POS // 8, 1.0, -1.0)
    t = s * sign
    ls = jnp.minimum(t, 0.0) - jnp.log(1.0 + jnp.exp(-jnp.abs(t)))
    o_ref[0, 0] = -jnp.sum(ls)


_tc_finish = pl.pallas_call(
    _tc_body,
    out_shape=jax.ShapeDtypeStruct((1, 1), jnp.float32),
    out_specs=pl.BlockSpec(memory_space=pltpu.SMEM),
)


def kernel(pos_u, pos_v, neg_u, neg_v, u_emb, v_emb):
    pvt, nvt = _prep_idx(pos_v.astype(jnp.int32).T, neg_v.astype(jnp.int32).T)
    accs = _sc_ctx(u_emb, pvt, nvt)
    parts = _sc_dot(v_emb, pos_u.astype(jnp.int32), neg_u.astype(jnp.int32),
                    accs)
    loss = _tc_finish(parts.reshape(R128, 128))
    return loss[0, 0]
